# baseline (speedup 1.0000x reference)
import functools

import jax
import jax.numpy as jnp
from jax import lax
from jax.experimental import pallas as pl
from jax.experimental.pallas import tpu as pltpu
from jax.experimental.pallas import tpu_sc as plsc

B, H, W, C = 4, 384, 384, 96
N = B * H * W

NUM_CORES = 2
NUM_SUBCORES = 16
NUM_TILES = NUM_CORES * NUM_SUBCORES
PER_TILE = N // NUM_TILES
P = 128
CHUNKS = PER_TILE // P
CU = 6

BR = 128


def _prep_body(fy_ref, fx_ref, idx_ref, w4_ref):
    g = pl.program_id(0) * BR + lax.broadcasted_iota(jnp.int32, (BR, W), 0)
    b = g // H
    i = g - b * H
    j = lax.broadcasted_iota(jnp.int32, (BR, W), 1)
    qy = i.astype(jnp.float32) - fy_ref[...]
    qx = j.astype(jnp.float32) - fx_ref[...]
    fy_f = jnp.clip(jnp.floor(qy), 0.0, float(H - 2))
    fx_f = jnp.clip(jnp.floor(qx), 0.0, float(W - 2))
    y0 = fy_f.astype(jnp.int32)
    x0 = fx_f.astype(jnp.int32)
    ay = jnp.clip(qy - fy_f, 0.0, 1.0)
    ax = jnp.clip(qx - fx_f, 0.0, 1.0)
    idx_ref[...] = (b * H + y0) * W + x0
    by = 1.0 - ay
    bx = 1.0 - ax
    w4_ref[0] = by * bx
    w4_ref[1] = by * ax
    w4_ref[2] = ay * bx
    w4_ref[3] = ay * ax


def _prep(fy, fx):
    BH = B * H
    return pl.pallas_call(
        _prep_body,
        grid=(BH // BR,),
        in_specs=[
            pl.BlockSpec((BR, W), lambda i: (i, 0)),
            pl.BlockSpec((BR, W), lambda i: (i, 0)),
        ],
        out_specs=[
            pl.BlockSpec((BR, W), lambda i: (i, 0)),
            pl.BlockSpec((4, BR, W), lambda i: (0, i, 0)),
        ],
        out_shape=[
            jax.ShapeDtypeStruct((BH, W), jnp.int32),
            jax.ShapeDtypeStruct((4, BH, W), jnp.float32),
        ],
    )(fy, fx)


def _warp_sc_body(img_hbm, idx_hbm, w4_hbm, out_hbm,
                  i00_v, i01_v, i10_v, i11_v,
                  r00_v, r01_v, r10_v, r11_v,
                  w00_v, w01_v, w10_v, w11_v, out_v, sem):
    wid = lax.axis_index("s") * NUM_CORES + lax.axis_index("c")
    tile_base = wid * PER_TILE

    def chunk_body(ci, carry):
        base = tile_base + ci * P
        pltpu.sync_copy(idx_hbm.at[pl.ds(base, P)], i00_v)
        for g in range(P // 16):
            s = pl.ds(g * 16, 16)
            v = i00_v[s]
            i01_v[s] = v + 1
            i10_v[s] = v + W
            i11_v[s] = v + W + 1
        cp0 = pltpu.async_copy(img_hbm.at[i00_v], r00_v, sem)
        cp1 = pltpu.async_copy(img_hbm.at[i01_v], r01_v, sem)
        cp2 = pltpu.async_copy(img_hbm.at[i10_v], r10_v, sem)
        cp3 = pltpu.async_copy(img_hbm.at[i11_v], r11_v, sem)
        pltpu.sync_copy(w4_hbm.at[0, pl.ds(base, P)], w00_v)
        pltpu.sync_copy(w4_hbm.at[1, pl.ds(base, P)], w01_v)
        pltpu.sync_copy(w4_hbm.at[2, pl.ds(base, P)], w10_v)
        pltpu.sync_copy(w4_hbm.at[3, pl.ds(base, P)], w11_v)
        cp0.wait()
        cp1.wait()
        cp2.wait()
        cp3.wait()

        def px_body(p, cc):
            pcol = jnp.full((16,), p, jnp.int32)
            w00 = plsc.load_gather(w00_v, [pcol])
            w01 = plsc.load_gather(w01_v, [pcol])
            w10 = plsc.load_gather(w10_v, [pcol])
            w11 = plsc.load_gather(w11_v, [pcol])
            for u in range(C // 16):
                s = pl.ds(u * 16, 16)
                out_v[p, s] = (w00 * r00_v[p, s] + w01 * r01_v[p, s]
                               + w10 * r10_v[p, s] + w11 * r11_v[p, s])
            return cc

        lax.fori_loop(0, P, px_body, 0)
        pltpu.sync_copy(out_v, out_hbm.at[pl.ds(base, P)])
        return carry

    lax.fori_loop(0, CHUNKS, chunk_body, 0)


@functools.cache
def _make_warp_sc():
    mesh = plsc.VectorSubcoreMesh(
        core_axis_name="c", subcore_axis_name="s",
        num_cores=NUM_CORES, num_subcores=NUM_SUBCORES,
    )
    return pl.kernel(
        _warp_sc_body,
        out_type=jax.ShapeDtypeStruct((N, C), jnp.float32),
        mesh=mesh,
        compiler_params=pltpu.CompilerParams(
            needs_layout_passes=False, use_tc_tiling_on_sc=False),
        scratch_types=[
        pltpu.VMEM((P,), jnp.int32),
        pltpu.VMEM((P,), jnp.int32),
        pltpu.VMEM((P,), jnp.int32),
        pltpu.VMEM((P,), jnp.int32),
        pltpu.VMEM((P, C), jnp.float32),
        pltpu.VMEM((P, C), jnp.float32),
        pltpu.VMEM((P, C), jnp.float32),
        pltpu.VMEM((P, C), jnp.float32),
        pltpu.VMEM((P,), jnp.float32),
        pltpu.VMEM((P,), jnp.float32),
        pltpu.VMEM((P,), jnp.float32),
        pltpu.VMEM((P,), jnp.float32),
        pltpu.VMEM((P, C), jnp.float32),
        pltpu.SemaphoreType.DMA,
    ],
    )


def kernel(frame_tail, flow):
    fy = flow[..., 0].reshape(B * H, W)
    fx = flow[..., 1].reshape(B * H, W)
    idx, w4 = _prep(fy, fx)
    warp = _make_warp_sc()
    out = warp(frame_tail.reshape(N, C), idx.reshape(N), w4.reshape(4, N))
    return out.reshape(B, H, W, C)


# speedup vs baseline: 1.2611x; 1.2611x over previous
import functools

import jax
import jax.numpy as jnp
from jax import lax
from jax.experimental import pallas as pl
from jax.experimental.pallas import tpu as pltpu
from jax.experimental.pallas import tpu_sc as plsc

B, H, W, C = 4, 384, 384, 96
N = B * H * W

NUM_CORES = 2
NUM_SUBCORES = 16
NUM_TILES = NUM_CORES * NUM_SUBCORES
PER_TILE = N // NUM_TILES
P = 96
CHUNKS = PER_TILE // P


def _derive(base, fy_v, fx_v, i00_v, i01_v, i10_v, i11_v,
            w00_v, w01_v, w10_v, w11_v):
    lane = lax.broadcasted_iota(jnp.int32, (16,), 0)
    for g in range(P // 16):
        s = pl.ds(g * 16, 16)
        pix = base + g * 16 + lane
        row = pix // W
        j = pix - row * W
        i = lax.rem(row, H)
        qy = i.astype(jnp.float32) - fy_v[s]
        qx = j.astype(jnp.float32) - fx_v[s]
        qyc = jnp.clip(qy, 0.0, float(H - 2))
        qxc = jnp.clip(qx, 0.0, float(W - 2))
        y0 = qyc.astype(jnp.int32)
        x0 = qxc.astype(jnp.int32)
        ay = jnp.clip(qy - y0.astype(jnp.float32), 0.0, 1.0)
        ax = jnp.clip(qx - x0.astype(jnp.float32), 0.0, 1.0)
        idx = (row - i + y0) * W + x0
        i00_v[s] = idx
        i01_v[s] = idx + 1
        i10_v[s] = idx + W
        i11_v[s] = idx + W + 1
        by = 1.0 - ay
        bx = 1.0 - ax
        w00_v[s] = by * bx
        w01_v[s] = by * ax
        w10_v[s] = ay * bx
        w11_v[s] = ay * ax


def _blend(r00_v, r01_v, r10_v, r11_v, w00_v, w01_v, w10_v, w11_v, out_v):
    def px_body(p, cc):
        pcol = jnp.full((16,), p, jnp.int32)
        w00 = plsc.load_gather(w00_v, [pcol])
        w01 = plsc.load_gather(w01_v, [pcol])
        w10 = plsc.load_gather(w10_v, [pcol])
        w11 = plsc.load_gather(w11_v, [pcol])
        for u in range(C // 16):
            s = pl.ds(u * 16, 16)
            out_v[p, s] = (w00 * r00_v[p, s] + w01 * r01_v[p, s]
                           + w10 * r10_v[p, s] + w11 * r11_v[p, s])
        return cc

    lax.fori_loop(0, P, px_body, 0)


def _warp_sc_body(img_hbm, fy_hbm, fx_hbm, out_hbm, *refs):
    (fy0, fy1, fx0, fx1,
     i00a, i01a, i10a, i11a, i00b, i01b, i10b, i11b,
     w00a, w01a, w10a, w11a, w00b, w01b, w10b, w11b,
     r00a, r01a, r10a, r11a, r00b, r01b, r10b, r11b,
     outa, outb,
     sem_f0, sem_f1, sem_g0, sem_g1, sem_o0, sem_o1) = refs

    wid = lax.axis_index("s") * NUM_CORES + lax.axis_index("c")
    tile_base = wid * PER_TILE

    bufs = (
        (fy0, fx0, (i00a, i01a, i10a, i11a), (w00a, w01a, w10a, w11a),
         (r00a, r01a, r10a, r11a), outa, sem_f0, sem_g0, sem_o0),
        (fy1, fx1, (i00b, i01b, i10b, i11b), (w00b, w01b, w10b, w11b),
         (r00b, r01b, r10b, r11b), outb, sem_f1, sem_g1, sem_o1),
    )

    def issue_flow(k, bi):
        fy_v, fx_v = bufs[bi][0], bufs[bi][1]
        base = tile_base + k * P
        pltpu.async_copy(fy_hbm.at[pl.ds(base, P)], fy_v, bufs[bi][6])
        pltpu.async_copy(fx_hbm.at[pl.ds(base, P)], fx_v, bufs[bi][6])

    def wait_flow(bi):
        fy_v, fx_v = bufs[bi][0], bufs[bi][1]
        pltpu.make_async_copy(fy_hbm.at[pl.ds(0, P)], fy_v, bufs[bi][6]).wait()
        pltpu.make_async_copy(fx_hbm.at[pl.ds(0, P)], fx_v, bufs[bi][6]).wait()

    def derive_issue_gathers(k, bi):
        fy_v, fx_v, iv, wv, rv, _, _, sem_g, _ = bufs[bi]
        _derive(tile_base + k * P, fy_v, fx_v, *iv, *wv)
        pltpu.async_copy(img_hbm.at[iv[0]], rv[0], sem_g)
        pltpu.async_copy(img_hbm.at[iv[1]], rv[1], sem_g)
        pltpu.async_copy(img_hbm.at[iv[2]], rv[2], sem_g)
        pltpu.async_copy(img_hbm.at[iv[3]], rv[3], sem_g)

    def wait_gathers(bi):
        iv, rv, sem_g = bufs[bi][2], bufs[bi][4], bufs[bi][7]
        for q in range(4):
            pltpu.make_async_copy(img_hbm.at[iv[q]], rv[q], sem_g).wait()

    def issue_out(k, bi):
        out_v, sem_o = bufs[bi][5], bufs[bi][8]
        base = tile_base + k * P
        pltpu.async_copy(out_v, out_hbm.at[pl.ds(base, P)], sem_o)

    def wait_out(bi):
        out_v, sem_o = bufs[bi][5], bufs[bi][8]
        pltpu.make_async_copy(out_v, out_hbm.at[pl.ds(0, P)], sem_o).wait()

    def blend(bi):
        _, _, _, wv, rv, out_v, _, _, _ = bufs[bi]
        _blend(*rv, *wv, out_v)

    issue_flow(0, 0)
    wait_flow(0)
    derive_issue_gathers(0, 0)
    issue_flow(1, 1)

    def pair_body(kk, cc):
        k = kk * 2
        wait_flow(1)
        derive_issue_gathers(k + 1, 1)

        @pl.when(kk + 1 < CHUNKS // 2)
        def _():
            issue_flow(k + 2, 0)

        wait_gathers(0)

        @pl.when(kk >= 1)
        def _():
            wait_out(0)

        blend(0)
        issue_out(k, 0)

        @pl.when(kk + 1 < CHUNKS // 2)
        def _():
            wait_flow(0)
            derive_issue_gathers(k + 2, 0)

        @pl.when(kk + 1 < CHUNKS // 2)
        def _():
            issue_flow(k + 3, 1)

        wait_gathers(1)

        @pl.when(kk >= 1)
        def _():
            wait_out(1)

        blend(1)
        issue_out(k + 1, 1)
        return cc

    lax.fori_loop(0, CHUNKS // 2, pair_body, 0)
    wait_out(0)
    wait_out(1)


@functools.cache
def _make_warp_sc():
    mesh = plsc.VectorSubcoreMesh(
        core_axis_name="c", subcore_axis_name="s",
        num_cores=NUM_CORES, num_subcores=NUM_SUBCORES,
    )
    idx_t = pltpu.VMEM((P,), jnp.int32)
    wgt_t = pltpu.VMEM((P,), jnp.float32)
    row_t = pltpu.VMEM((P, C), jnp.float32)
    return pl.kernel(
        _warp_sc_body,
        out_type=jax.ShapeDtypeStruct((N, C), jnp.float32),
        mesh=mesh,
        compiler_params=pltpu.CompilerParams(
            needs_layout_passes=False, use_tc_tiling_on_sc=False),
        scratch_types=(
            [wgt_t] * 4
            + [idx_t] * 8
            + [wgt_t] * 8
            + [row_t] * 8
            + [row_t] * 2
            + [pltpu.SemaphoreType.DMA] * 6
        ),
    )


def kernel(frame_tail, flow):
    fy = flow[..., 0].reshape(N)
    fx = flow[..., 1].reshape(N)
    warp = _make_warp_sc()
    out = warp(frame_tail.reshape(N, C), fy, fx)
    return out.reshape(B, H, W, C)
